# Initial kernel scaffold; baseline (speedup 1.0000x reference)
import functools

import jax
import jax.numpy as jnp
from jax import lax
from jax.experimental import pallas as pl
from jax.experimental.pallas import tpu as pltpu
from jax.experimental.pallas import tpu_sc as plsc

_M, _K, _D = 8, 8192, 32
_N, _H, _W = 16, 32, 32
_HW = _H * _W
_ROWS = _N * _H * _W * _M
_NW = 32
_CROWS = 128
_CHUNKS = _ROWS // (_NW * _CROWS)
_GRP = 8


def _pcb_body(cb_ref, wq_ref, bq_ref, out_ref):
    cb = cb_ref[0]
    w = wq_ref[0]
    out_ref[...] = lax.dot_general(
        cb, w, (((1,), (1,)), ((), ())),
        preferred_element_type=jnp.float32) + bq_ref[0]


def _compute_pcb(codebook, wq, bq):
    return pl.pallas_call(
        _pcb_body,
        grid=(_M,),
        in_specs=[
            pl.BlockSpec((1, _K, _D), lambda m: (m, 0, 0)),
            pl.BlockSpec((1, _D, _D), lambda m: (m, 0, 0)),
            pl.BlockSpec((1, 1, _D), lambda m: (m, 0, 0)),
        ],
        out_specs=pl.BlockSpec((_K, _D), lambda m: (m, 0)),
        out_shape=jax.ShapeDtypeStruct((_M * _K, _D), jnp.float32),
    )(codebook, wq, bq.reshape(_M, 1, _D))


def _gather_sc(table, idx):
    info = plsc.get_sparse_core_info()
    nc = info.num_cores
    mesh = plsc.VectorSubcoreMesh(core_axis_name="c", subcore_axis_name="s")

    @functools.partial(
        pl.kernel,
        mesh=mesh,
        out_type=jax.ShapeDtypeStruct((_NW * _CHUNKS, _CROWS, _D),
                                      jnp.float32),
        scratch_types=[
            pltpu.VMEM((_CHUNKS, _CROWS), jnp.int32),
            pltpu.VMEM((_GRP, _CROWS, _D), jnp.float32),
            pltpu.VMEM((_GRP, _CROWS, _D), jnp.float32),
            pltpu.SemaphoreType.DMA,
            pltpu.SemaphoreType.DMA,
        ],
    )
    def gk(table_hbm, idx_hbm, out_hbm, idx_v, bufa, bufb, sema, semb):
        wid = lax.axis_index("s") * nc + lax.axis_index("c")
        pltpu.sync_copy(idx_hbm.at[wid], idx_v)
        bufs = (bufa, bufb)
        sems = (sema, semb)
        ngrp = _CHUNKS // _GRP

        def issue(g):
            buf, sem = bufs[g % 2], sems[g % 2]
            return [
                pltpu.async_copy(table_hbm.at[idx_v.at[g * _GRP + j]],
                                 buf.at[j], sem)
                for j in range(_GRP)
            ]

        cps = issue(0)
        for g in range(ngrp):
            nxt = issue(g + 1) if g + 1 < ngrp else None
            for cp in cps:
                cp.wait()
            pltpu.sync_copy(bufs[g % 2],
                            out_hbm.at[pl.ds(wid * _CHUNKS + g * _GRP, _GRP)])
            cps = nxt

    return gk(table, idx)


def _tr_body(g_ref, o_ref):
    o_ref[0] = jnp.transpose(g_ref[0], (1, 0))


def _transpose(g):
    nm = _N * _M
    return pl.pallas_call(
        _tr_body,
        grid=(nm,),
        in_specs=[pl.BlockSpec((1, _HW, _D), lambda i: (i, 0, 0))],
        out_specs=pl.BlockSpec((1, _D, _HW), lambda i: (i, 0, 0)),
        out_shape=jax.ShapeDtypeStruct((nm, _D, _HW), jnp.float32),
    )(g)


def kernel(codes, codebook, wq, bq):
    pcb = _compute_pcb(codebook, wq, bq)
    offs = jnp.arange(_M, dtype=jnp.int32) * _K
    idx = codes.transpose(0, 3, 1, 2).reshape(_N, _M, _HW) + offs[None, :, None]
    idx = idx.reshape(_NW, _CHUNKS, _CROWS)
    g = _gather_sc(pcb, idx)
    out = _transpose(g.reshape(_N * _M, _HW, _D))
    return out.reshape(_N, _M * _D, _H, _W)


# speedup vs baseline: 8.5425x; 8.5425x over previous
import functools

import jax
import jax.numpy as jnp
from jax import lax
from jax.experimental import pallas as pl
from jax.experimental.pallas import tpu as pltpu
from jax.experimental.pallas import tpu_sc as plsc

_M, _K, _D = 8, 8192, 32
_N, _H, _W = 16, 32, 32
_HW = _H * _W
_ROWS = _N * _H * _W * _M
_NW = 32
_CROWS = 128
_CHUNKS = _ROWS // (_NW * _CROWS)
_GRP = 8


def _pcb_body(cb_ref, wq_ref, bq_ref, out_ref):
    cb = cb_ref[0]
    w = wq_ref[0]
    out_ref[...] = lax.dot_general(
        cb, w, (((1,), (1,)), ((), ())),
        preferred_element_type=jnp.float32) + bq_ref[0]


def _compute_pcb(codebook, wq, bq):
    return pl.pallas_call(
        _pcb_body,
        grid=(_M,),
        in_specs=[
            pl.BlockSpec((1, _K, _D), lambda m: (m, 0, 0)),
            pl.BlockSpec((1, _D, _D), lambda m: (m, 0, 0)),
            pl.BlockSpec((1, 1, _D), lambda m: (m, 0, 0)),
        ],
        out_specs=pl.BlockSpec((_K, _D), lambda m: (m, 0)),
        out_shape=jax.ShapeDtypeStruct((_M * _K, _D), jnp.float32),
    )(codebook, wq, bq.reshape(_M, 1, _D))


def _gather_sc(table, idx):
    info = plsc.get_sparse_core_info()
    nc = info.num_cores
    mesh = plsc.VectorSubcoreMesh(core_axis_name="c", subcore_axis_name="s")

    @functools.partial(
        pl.kernel,
        mesh=mesh,
        compiler_params=pltpu.CompilerParams(use_tc_tiling_on_sc=False),
        out_type=jax.ShapeDtypeStruct((_NW * _CHUNKS, _CROWS, _D),
                                      jnp.float32),
        scratch_types=[
            pltpu.VMEM((_CHUNKS, _CROWS), jnp.int32),
            pltpu.VMEM((_GRP, _CROWS, _D), jnp.float32),
            pltpu.VMEM((_GRP, _CROWS, _D), jnp.float32),
            pltpu.SemaphoreType.DMA,
            pltpu.SemaphoreType.DMA,
        ],
    )
    def gk(table_hbm, idx_hbm, out_hbm, idx_v, bufa, bufb, sema, semb):
        wid = lax.axis_index("s") * nc + lax.axis_index("c")
        pltpu.sync_copy(idx_hbm.at[wid], idx_v)
        bufs = (bufa, bufb)
        sems = (sema, semb)
        ngrp = _CHUNKS // _GRP

        def issue(g):
            buf, sem = bufs[g % 2], sems[g % 2]
            return [
                pltpu.async_copy(table_hbm.at[idx_v.at[g * _GRP + j]],
                                 buf.at[j], sem)
                for j in range(_GRP)
            ]

        cps = issue(0)
        for g in range(ngrp):
            nxt = issue(g + 1) if g + 1 < ngrp else None
            for cp in cps:
                cp.wait()
            pltpu.sync_copy(bufs[g % 2],
                            out_hbm.at[pl.ds(wid * _CHUNKS + g * _GRP, _GRP)])
            cps = nxt

    return gk(table, idx)


def _tr_body(g_ref, o_ref):
    o_ref[0] = jnp.transpose(g_ref[0], (1, 0))


def _transpose(g):
    nm = _N * _M
    return pl.pallas_call(
        _tr_body,
        grid=(nm,),
        in_specs=[pl.BlockSpec((1, _HW, _D), lambda i: (i, 0, 0))],
        out_specs=pl.BlockSpec((1, _D, _HW), lambda i: (i, 0, 0)),
        out_shape=jax.ShapeDtypeStruct((nm, _D, _HW), jnp.float32),
    )(g)


def kernel(codes, codebook, wq, bq):
    pcb = _compute_pcb(codebook, wq, bq)
    offs = jnp.arange(_M, dtype=jnp.int32) * _K
    idx = codes.transpose(0, 3, 1, 2).reshape(_N, _M, _HW) + offs[None, :, None]
    idx = idx.reshape(_NW, _CHUNKS, _CROWS)
    g = _gather_sc(pcb, idx)
    out = _transpose(g.reshape(_N * _M, _HW, _D))
    return out.reshape(_N, _M * _D, _H, _W)
